# baseline (speedup 1.0000x reference)
import jax
import jax.numpy as jnp
from jax.experimental import pallas as pl
from jax.experimental.pallas import tpu as pltpu

_B, _S, _D, _E, _K = 4, 4096, 4096, 64, 8
_T = 1024
_NBUF = 3
_NSTRIPE = 2
_TS = _T // _NSTRIPE


def _router_kernel(x_hbm, w_ref, b_ref, logits_ref, weights_ref,
                   experts_ref, xbuf, sems):
    i = pl.program_id(0)
    nsteps = pl.num_programs(0)

    def start_copy(c):
        slot = jax.lax.rem(c, _NBUF)
        for s in range(_NSTRIPE):
            pltpu.make_async_copy(
                x_hbm.at[pl.ds(c * _T + s * _TS, _TS), :],
                xbuf.at[slot, pl.ds(s * _TS, _TS), :],
                sems.at[slot, s],
            ).start()

    def wait_copy(c):
        slot = jax.lax.rem(c, _NBUF)
        for s in range(_NSTRIPE):
            pltpu.make_async_copy(
                x_hbm.at[pl.ds(c * _T + s * _TS, _TS), :],
                xbuf.at[slot, pl.ds(s * _TS, _TS), :],
                sems.at[slot, s],
            ).wait()

    @pl.when(i == 0)
    def _():
        for j in range(_NBUF - 1):
            @pl.when(jnp.int32(j) < nsteps)
            def _():
                start_copy(jnp.int32(j))

    @pl.when(i + (_NBUF - 1) < nsteps)
    def _():
        start_copy(i + (_NBUF - 1))

    slot = jax.lax.rem(i, _NBUF)
    wait_copy(i)

    x = xbuf[slot]
    w = w_ref[...]
    logits = jnp.dot(x, w, preferred_element_type=jnp.float32)
    logits = logits + b_ref[...]
    logits_ref[...] = logits

    lt = logits.T
    bits = jax.lax.bitcast_convert_type(lt, jnp.int32)
    key = jnp.where(bits < 0, bits ^ jnp.int32(0x7FFFFFFF), bits)
    row = jax.lax.broadcasted_iota(jnp.int32, lt.shape, 0)
    packed = (key & jnp.int32(~0x3F)) | (jnp.int32(_E - 1) - row)

    kiota = jax.lax.broadcasted_iota(jnp.int32, (_K, lt.shape[1]), 0)
    top_vals = jnp.zeros((_K, lt.shape[1]), jnp.float32)
    top_idx = jnp.zeros((_K, lt.shape[1]), jnp.int32)
    cur = packed
    imin = jnp.int32(-0x80000000)
    for k in range(_K):
        m = jnp.max(cur, axis=0, keepdims=True)
        cur = jnp.where(cur == m, imin, cur)
        idx_k = jnp.int32(_E - 1) - (m & jnp.int32(0x3F))
        keybits = m | jnp.int32(0x3F)
        vbits = jnp.where(keybits < 0, keybits ^ jnp.int32(0x7FFFFFFF), keybits)
        val_k = jax.lax.bitcast_convert_type(vbits, jnp.float32)
        top_idx = jnp.where(kiota == k, idx_k, top_idx)
        top_vals = jnp.where(kiota == k, val_k, top_vals)

    e = jnp.exp(top_vals - top_vals[0:1, :])
    weights_ref[...] = e / jnp.sum(e, axis=0, keepdims=True)
    experts_ref[...] = top_idx


def kernel(hidden_states, gate_w, gate_b):
    B, S, D = hidden_states.shape
    E = gate_w.shape[0]
    N = B * S
    x = hidden_states.reshape(N, D)
    wt = gate_w.T
    b = gate_b.reshape(1, E)

    grid = (N // _T,)
    logits, weights_t, experts_t = pl.pallas_call(
        _router_kernel,
        grid=grid,
        in_specs=[
            pl.BlockSpec(memory_space=pltpu.MemorySpace.HBM),
            pl.BlockSpec((D, E), lambda i: (0, 0)),
            pl.BlockSpec((1, E), lambda i: (0, 0)),
        ],
        out_specs=[
            pl.BlockSpec((_T, E), lambda i: (i, 0)),
            pl.BlockSpec((_K, _T), lambda i: (0, i)),
            pl.BlockSpec((_K, _T), lambda i: (0, i)),
        ],
        out_shape=[
            jax.ShapeDtypeStruct((N, E), jnp.float32),
            jax.ShapeDtypeStruct((_K, N), jnp.float32),
            jax.ShapeDtypeStruct((_K, N), jnp.int32),
        ],
        scratch_shapes=[
            pltpu.VMEM((_NBUF, _T, D), jnp.float32),
            pltpu.SemaphoreType.DMA((_NBUF, _NSTRIPE)),
        ],
    )(x, wt, b)

    return (weights_t.T.reshape(B, S, _K),
            experts_t.T.reshape(B, S, _K),
            logits.reshape(B, S, E))


# speedup vs baseline: 1.0027x; 1.0027x over previous
import jax
import jax.numpy as jnp
from jax.experimental import pallas as pl
from jax.experimental.pallas import tpu as pltpu

_B, _S, _D, _E, _K = 4, 4096, 4096, 64, 8
_T = 256
_RING_ROWS = 3072
_NSEM = 4
_LOOKAHEAD = 2


def _chunk_schedule(n_rows):
    ramp = [256, 256, 512]
    tail = [512, 256, 256]
    mid = n_rows - sum(ramp) - sum(tail)
    if mid >= 0 and mid % 1024 == 0:
        sizes = ramp + [1024] * (mid // 1024) + tail
    else:
        sizes = [_T] * (n_rows // _T)
    chunks = []
    row = 0
    for c, rows in enumerate(sizes):
        chunks.append((row, row % _RING_ROWS, rows, c % _NSEM, row // _T))
        row += rows
    return chunks


def _make_router_kernel(n_rows, d):
    chunks = _chunk_schedule(n_rows)

    def body(x_hbm, w_ref, b_ref, logits_ref, weights_ref, experts_ref,
             ring, sems):
        i = pl.program_id(0)

        def copy(c):
            src, dst, rows, slot, _ = chunks[c]
            return pltpu.make_async_copy(
                x_hbm.at[pl.ds(src, rows), :],
                ring.at[pl.ds(dst, rows), :],
                sems.at[slot],
            )

        for c in range(len(chunks)):
            t_start = 0 if c <= _LOOKAHEAD else chunks[c - _LOOKAHEAD][4]

            @pl.when(i == t_start)
            def _(c=c):
                copy(c).start()

        for c, (_, _, _, _, first_tile) in enumerate(chunks):
            @pl.when(i == first_tile)
            def _(c=c):
                copy(c).wait()

        off = jax.lax.rem(i * _T, _RING_ROWS)
        x = ring[pl.ds(off, _T), :]
        w = w_ref[...]
        logits = jnp.dot(x, w, preferred_element_type=jnp.float32)
        logits = logits + b_ref[...]
        logits_ref[...] = logits

        lt = logits.T
        bits = jax.lax.bitcast_convert_type(lt, jnp.int32)
        key = jnp.where(bits < 0, bits ^ jnp.int32(0x7FFFFFFF), bits)
        row = jax.lax.broadcasted_iota(jnp.int32, lt.shape, 0)
        packed = (key & jnp.int32(~0x3F)) | (jnp.int32(_E - 1) - row)

        kiota = jax.lax.broadcasted_iota(jnp.int32, (_K, lt.shape[1]), 0)
        top_vals = jnp.zeros((_K, lt.shape[1]), jnp.float32)
        top_idx = jnp.zeros((_K, lt.shape[1]), jnp.int32)
        cur = packed
        imin = jnp.int32(-0x80000000)
        for k in range(_K):
            m = jnp.max(cur, axis=0, keepdims=True)
            cur = jnp.where(cur == m, imin, cur)
            idx_k = jnp.int32(_E - 1) - (m & jnp.int32(0x3F))
            keybits = m | jnp.int32(0x3F)
            vbits = jnp.where(keybits < 0,
                              keybits ^ jnp.int32(0x7FFFFFFF), keybits)
            val_k = jax.lax.bitcast_convert_type(vbits, jnp.float32)
            top_idx = jnp.where(kiota == k, idx_k, top_idx)
            top_vals = jnp.where(kiota == k, val_k, top_vals)

        e = jnp.exp(top_vals - top_vals[0:1, :])
        weights_ref[...] = e / jnp.sum(e, axis=0, keepdims=True)
        experts_ref[...] = top_idx

    return body


def kernel(hidden_states, gate_w, gate_b):
    B, S, D = hidden_states.shape
    E = gate_w.shape[0]
    N = B * S
    x = hidden_states.reshape(N, D)
    wt = gate_w.T
    b = gate_b.reshape(1, E)

    grid = (N // _T,)
    logits, weights_t, experts_t = pl.pallas_call(
        _make_router_kernel(N, D),
        grid=grid,
        in_specs=[
            pl.BlockSpec(memory_space=pltpu.MemorySpace.HBM),
            pl.BlockSpec((D, E), lambda i: (0, 0)),
            pl.BlockSpec((1, E), lambda i: (0, 0)),
        ],
        out_specs=[
            pl.BlockSpec((_T, E), lambda i: (i, 0)),
            pl.BlockSpec((_K, _T), lambda i: (0, i)),
            pl.BlockSpec((_K, _T), lambda i: (0, i)),
        ],
        out_shape=[
            jax.ShapeDtypeStruct((N, E), jnp.float32),
            jax.ShapeDtypeStruct((_K, N), jnp.float32),
            jax.ShapeDtypeStruct((_K, N), jnp.int32),
        ],
        scratch_shapes=[
            pltpu.VMEM((_RING_ROWS, D), jnp.float32),
            pltpu.SemaphoreType.DMA((_NSEM,)),
        ],
    )(x, wt, b)

    return (weights_t.T.reshape(B, S, _K),
            experts_t.T.reshape(B, S, _K),
            logits.reshape(B, S, E))


# speedup vs baseline: 1.0041x; 1.0014x over previous
import jax
import jax.numpy as jnp
from jax.experimental import pallas as pl
from jax.experimental.pallas import tpu as pltpu

_B, _S, _D, _E, _K = 4, 4096, 4096, 64, 8
_T = 512
_RING_ROWS = 3072
_NSEM = 4
_LOOKAHEAD = 2


def _chunk_schedule(n_rows):
    ramp = [512, 512]
    tail = [512, 512]
    mid = n_rows - sum(ramp) - sum(tail)
    if mid >= 0 and mid % 1024 == 0:
        sizes = ramp + [1024] * (mid // 1024) + tail
    else:
        sizes = [_T] * (n_rows // _T)
    chunks = []
    row = 0
    for c, rows in enumerate(sizes):
        chunks.append((row, row % _RING_ROWS, rows, c % _NSEM, row // _T))
        row += rows
    return chunks


def _make_router_kernel(n_rows, d):
    chunks = _chunk_schedule(n_rows)

    def body(x_hbm, w_ref, b_ref, logits_ref, weights_ref, experts_ref,
             ring, sems):
        i = pl.program_id(0)

        def copy(c):
            src, dst, rows, slot, _ = chunks[c]
            return pltpu.make_async_copy(
                x_hbm.at[pl.ds(src, rows), :],
                ring.at[pl.ds(dst, rows), :],
                sems.at[slot],
            )

        for c in range(len(chunks)):
            t_start = 0 if c <= _LOOKAHEAD else chunks[c - _LOOKAHEAD][4]

            @pl.when(i == t_start)
            def _(c=c):
                copy(c).start()

        for c, (_, _, _, _, first_tile) in enumerate(chunks):
            @pl.when(i == first_tile)
            def _(c=c):
                copy(c).wait()

        off = jax.lax.rem(i * _T, _RING_ROWS)
        x = ring[pl.ds(off, _T), :]
        w = w_ref[...]
        logits = jnp.dot(x, w, preferred_element_type=jnp.float32)
        logits = logits + b_ref[...]
        logits_ref[...] = logits

        lt = logits.T
        bits = jax.lax.bitcast_convert_type(lt, jnp.int32)
        key = jnp.where(bits < 0, bits ^ jnp.int32(0x7FFFFFFF), bits)
        row = jax.lax.broadcasted_iota(jnp.int32, lt.shape, 0)
        packed = (key & jnp.int32(~0x3F)) | (jnp.int32(_E - 1) - row)

        kiota = jax.lax.broadcasted_iota(jnp.int32, (_K, lt.shape[1]), 0)
        top_vals = jnp.zeros((_K, lt.shape[1]), jnp.float32)
        top_idx = jnp.zeros((_K, lt.shape[1]), jnp.int32)
        cur = packed
        imin = jnp.int32(-0x80000000)
        for k in range(_K):
            m = jnp.max(cur, axis=0, keepdims=True)
            cur = jnp.where(cur == m, imin, cur)
            idx_k = jnp.int32(_E - 1) - (m & jnp.int32(0x3F))
            keybits = m | jnp.int32(0x3F)
            vbits = jnp.where(keybits < 0,
                              keybits ^ jnp.int32(0x7FFFFFFF), keybits)
            val_k = jax.lax.bitcast_convert_type(vbits, jnp.float32)
            top_idx = jnp.where(kiota == k, idx_k, top_idx)
            top_vals = jnp.where(kiota == k, val_k, top_vals)

        e = jnp.exp(top_vals - top_vals[0:1, :])
        weights_ref[...] = e / jnp.sum(e, axis=0, keepdims=True)
        experts_ref[...] = top_idx

    return body


def kernel(hidden_states, gate_w, gate_b):
    B, S, D = hidden_states.shape
    E = gate_w.shape[0]
    N = B * S
    x = hidden_states.reshape(N, D)
    wt = gate_w.T
    b = gate_b.reshape(1, E)

    grid = (N // _T,)
    logits, weights_t, experts_t = pl.pallas_call(
        _make_router_kernel(N, D),
        grid=grid,
        in_specs=[
            pl.BlockSpec(memory_space=pltpu.MemorySpace.HBM),
            pl.BlockSpec((D, E), lambda i: (0, 0)),
            pl.BlockSpec((1, E), lambda i: (0, 0)),
        ],
        out_specs=[
            pl.BlockSpec((_T, E), lambda i: (i, 0)),
            pl.BlockSpec((_K, _T), lambda i: (0, i)),
            pl.BlockSpec((_K, _T), lambda i: (0, i)),
        ],
        out_shape=[
            jax.ShapeDtypeStruct((N, E), jnp.float32),
            jax.ShapeDtypeStruct((_K, N), jnp.float32),
            jax.ShapeDtypeStruct((_K, N), jnp.int32),
        ],
        scratch_shapes=[
            pltpu.VMEM((_RING_ROWS, D), jnp.float32),
            pltpu.SemaphoreType.DMA((_NSEM,)),
        ],
    )(x, wt, b)

    return (weights_t.T.reshape(B, S, _K),
            experts_t.T.reshape(B, S, _K),
            logits.reshape(B, S, E))


# speedup vs baseline: 1.0274x; 1.0232x over previous
import jax
import jax.numpy as jnp
from jax.experimental import pallas as pl

_B, _S, _D, _E, _K = 4, 4096, 4096, 64, 8
_T = 1024


def _router_kernel(x_ref, w_ref, b_ref, logits_ref, weights_ref,
                   experts_ref):
    x = x_ref[...]
    w = w_ref[...]
    logits = jnp.dot(x, w, preferred_element_type=jnp.float32)
    logits = logits + b_ref[...]
    logits_ref[...] = logits

    lt = logits.T
    bits = jax.lax.bitcast_convert_type(lt, jnp.int32)
    key = jnp.where(bits < 0, bits ^ jnp.int32(0x7FFFFFFF), bits)
    row = jax.lax.broadcasted_iota(jnp.int32, lt.shape, 0)
    packed = (key & jnp.int32(~0x3F)) | (jnp.int32(_E - 1) - row)

    kiota = jax.lax.broadcasted_iota(jnp.int32, (_K, lt.shape[1]), 0)
    top_vals = jnp.zeros((_K, lt.shape[1]), jnp.float32)
    top_idx = jnp.zeros((_K, lt.shape[1]), jnp.int32)
    cur = packed
    imin = jnp.int32(-0x80000000)
    for k in range(_K):
        m = jnp.max(cur, axis=0, keepdims=True)
        cur = jnp.where(cur == m, imin, cur)
        idx_k = jnp.int32(_E - 1) - (m & jnp.int32(0x3F))
        keybits = m | jnp.int32(0x3F)
        vbits = jnp.where(keybits < 0, keybits ^ jnp.int32(0x7FFFFFFF), keybits)
        val_k = jax.lax.bitcast_convert_type(vbits, jnp.float32)
        top_idx = jnp.where(kiota == k, idx_k, top_idx)
        top_vals = jnp.where(kiota == k, val_k, top_vals)

    e = jnp.exp(top_vals - top_vals[0:1, :])
    weights_ref[...] = e / jnp.sum(e, axis=0, keepdims=True)
    experts_ref[...] = top_idx


def kernel(hidden_states, gate_w, gate_b):
    B, S, D = hidden_states.shape
    E = gate_w.shape[0]
    N = B * S
    x = hidden_states.reshape(N, D)
    wt = gate_w.T
    b = gate_b.reshape(1, E)

    grid = (N // _T,)
    logits, weights_t, experts_t = pl.pallas_call(
        _router_kernel,
        grid=grid,
        in_specs=[
            pl.BlockSpec((_T, D), lambda i: (i, 0)),
            pl.BlockSpec((D, E), lambda i: (0, 0)),
            pl.BlockSpec((1, E), lambda i: (0, 0)),
        ],
        out_specs=[
            pl.BlockSpec((_T, E), lambda i: (i, 0)),
            pl.BlockSpec((_K, _T), lambda i: (0, i)),
            pl.BlockSpec((_K, _T), lambda i: (0, i)),
        ],
        out_shape=[
            jax.ShapeDtypeStruct((N, E), jnp.float32),
            jax.ShapeDtypeStruct((_K, N), jnp.float32),
            jax.ShapeDtypeStruct((_K, N), jnp.int32),
        ],
    )(x, wt, b)

    return (weights_t.T.reshape(B, S, _K),
            experts_t.T.reshape(B, S, _K),
            logits.reshape(B, S, E))
